# baseline (speedup 1.0000x reference)
import functools

import jax
import jax.numpy as jnp
from jax import lax
from jax.experimental import pallas as pl
from jax.experimental.pallas import tpu as pltpu
from jax.experimental.pallas import tpu_sc as plsc

N = 10000
E = 320000
D = 128
NP = 10240
NC = 2
NS = 16
NW = NC * NS
EPW = E // NW
CH = 80
NCHUNK = EPW // CH
CHB = 128
NCHB = EPW // CHB
TAILB = EPW - NCHB * CHB
RPT = NP // NS
NB = 4
NBB = 2


def _sc_mesh():
    return plsc.VectorSubcoreMesh(core_axis_name="c", subcore_axis_name="s")


def _deg_body(dst_hbm, zeros_hbm, degp_hbm,
              di0, di1, di2, di3, ones_v, acc_sh,
              is0, is1, is2, is3, ss0, ss1, ss2, ss3):
    cid = lax.axis_index("c")
    sid = lax.axis_index("s")
    wid = cid * NS + sid
    r0 = sid * RPT
    dib = (di0, di1, di2, di3)
    isem = (is0, is1, is2, is3)
    ssem = (ss0, ss1, ss2, ss3)
    for k in range(CH // 16):
        ones_v[pl.ds(k * 16, 16)] = jnp.ones((16,), jnp.float32)
    pltpu.sync_copy(zeros_hbm.at[pl.ds(r0, RPT)], acc_sh.at[pl.ds(r0, RPT)])
    plsc.subcore_barrier()

    base = wid * EPW

    def group(k, carry):
        c0 = k * NB
        idescs = [
            pltpu.async_copy(
                dst_hbm.at[pl.ds(pl.multiple_of(base + (c0 + b) * CH, 8), CH)],
                dib[b], isem[b])
            for b in range(NB)
        ]
        sdescs = []
        for b in range(NB):
            idescs[b].wait()
            sdescs.append(
                pltpu.async_copy(ones_v, acc_sh.at[dib[b]], ssem[b], add=True))
        for d in sdescs:
            d.wait()
        return carry

    lax.fori_loop(0, NCHUNK // NB, group, 0)
    for c in range(NCHUNK - NCHUNK % NB, NCHUNK):
        off = pl.multiple_of(base + c * CH, 8)
        pltpu.async_copy(dst_hbm.at[pl.ds(off, CH)], di0, is0).wait()
        pltpu.async_copy(ones_v, acc_sh.at[di0], ss0, add=True).wait()
    plsc.subcore_barrier()
    pltpu.sync_copy(acc_sh.at[pl.ds(r0, RPT)], degp_hbm.at[cid, pl.ds(r0, RPT)])


_deg_call = functools.partial(
    pl.kernel,
    out_type=jax.ShapeDtypeStruct((NC, NP), jnp.float32),
    mesh=_sc_mesh(),
    scratch_types=(
        [pltpu.VMEM((CH,), jnp.int32) for _ in range(4)]
        + [pltpu.VMEM((CH,), jnp.float32)]
        + [pltpu.VMEM_SHARED((NP,), jnp.float32)]
        + [pltpu.SemaphoreType.DMA for _ in range(8)]
    ),
)(_deg_body)


def _agg_body(g_hbm, src_hbm, dst_hbm, zrow_hbm, tp_hbm, *rest):
    sib = rest[0:NBB]
    dib = rest[NBB:2 * NBB]
    sit, dit = rest[2 * NBB], rest[2 * NBB + 1]
    rows = rest[2 * NBB + 2:3 * NBB + 2]
    acc_sh = rest[3 * NBB + 2]
    isem = rest[3 * NBB + 3:4 * NBB + 3]
    gsem = rest[4 * NBB + 3:5 * NBB + 3]
    ssem = rest[5 * NBB + 3:6 * NBB + 3]
    cid = lax.axis_index("c")
    sid = lax.axis_index("s")
    wid = cid * NS + sid
    r0 = sid * RPT

    @pl.when(jnp.logical_and(cid == 0, sid < NS - 1))
    def _():
        pltpu.sync_copy(g_hbm.at[pl.ds(r0, RPT), :], acc_sh.at[pl.ds(r0, RPT), :])

    @pl.when(jnp.logical_and(cid == 0, sid == NS - 1))
    def _():
        pltpu.sync_copy(g_hbm.at[pl.ds(r0, N - r0), :],
                        acc_sh.at[pl.ds(r0, N - r0), :])
        pltpu.sync_copy(zrow_hbm.at[pl.ds(0, NP - N), :],
                        acc_sh.at[pl.ds(N, NP - N), :])

    @pl.when(cid != 0)
    def _():
        pltpu.sync_copy(zrow_hbm, acc_sh.at[pl.ds(r0, RPT), :])

    plsc.subcore_barrier()

    base = wid * EPW

    def drain_scatter(b):
        pltpu.make_async_copy(rows[b], acc_sh.at[dib[b]], ssem[b]).wait()

    def group(k, carry):
        c0 = k * NBB
        idescs = []
        for b in range(NBB):
            @pl.when(k > 0)
            def _():
                drain_scatter(b)
            off = pl.multiple_of(base + (c0 + b) * CHB, 8)
            idescs.append(
                (pltpu.async_copy(src_hbm.at[pl.ds(off, CHB)], sib[b], isem[b]),
                 pltpu.async_copy(dst_hbm.at[pl.ds(off, CHB)], dib[b], isem[b])))
        gdescs = []
        for b in range(NBB):
            idescs[b][0].wait()
            idescs[b][1].wait()
            gdescs.append(
                pltpu.async_copy(g_hbm.at[sib[b]], rows[b], gsem[b]))
        for b in range(NBB):
            gdescs[b].wait()
            pltpu.async_copy(rows[b], acc_sh.at[dib[b]], ssem[b], add=True)
        return carry

    lax.fori_loop(0, NCHB // NBB, group, 0)
    for b in range(NBB):
        drain_scatter(b)
    toff = pl.multiple_of(base + NCHB * CHB, 8)
    pltpu.async_copy(src_hbm.at[pl.ds(toff, TAILB)], sit, isem[0]).wait()
    pltpu.async_copy(dst_hbm.at[pl.ds(toff, TAILB)], dit, isem[0]).wait()
    pltpu.async_copy(g_hbm.at[sit], rows[0].at[pl.ds(0, TAILB), :],
                     gsem[0]).wait()
    pltpu.sync_copy(rows[0].at[pl.ds(0, TAILB), :], acc_sh.at[dit], add=True)

    plsc.subcore_barrier()
    pltpu.sync_copy(acc_sh.at[pl.ds(r0, RPT), :],
                    tp_hbm.at[cid, pl.ds(r0, RPT), :])


_agg_call = functools.partial(
    pl.kernel,
    out_type=jax.ShapeDtypeStruct((NC, NP, D), jnp.float32),
    mesh=_sc_mesh(),
    scratch_types=(
        [pltpu.VMEM((CHB,), jnp.int32) for _ in range(2 * NBB)]
        + [pltpu.VMEM((TAILB,), jnp.int32) for _ in range(2)]
        + [pltpu.VMEM((CHB, D), jnp.float32) for _ in range(NBB)]
        + [pltpu.VMEM_SHARED((NP, D), jnp.float32)]
        + [pltpu.SemaphoreType.DMA for _ in range(3 * NBB)]
    ),
)(_agg_body)


BR = 2000
GRID = N // BR


def _mm_body(x_ref, w_ref, h_ref):
    h_ref[...] = jnp.dot(x_ref[...], w_ref[...],
                         preferred_element_type=jnp.float32)


def _tc_matmul(x, W):
    return pl.pallas_call(
        _mm_body,
        grid=(GRID,),
        in_specs=[
            pl.BlockSpec((BR, D), lambda i: (i, 0)),
            pl.BlockSpec((D, D), lambda i: (0, 0)),
        ],
        out_specs=pl.BlockSpec((BR, D), lambda i: (i, 0)),
        out_shape=jax.ShapeDtypeStruct((NP, D), jnp.float32),
    )(x, W)


def _lin_body(h_ref, degp_ref, g_ref, dinv_ref):
    deg = jnp.sum(degp_ref[...], axis=0) + 1.0
    dinv = lax.rsqrt(deg)
    g_ref[...] = h_ref[...] * dinv
    dinv_ref[...] = dinv


def _tc_scale(h, degp_col):
    return pl.pallas_call(
        _lin_body,
        grid=(GRID,),
        in_specs=[
            pl.BlockSpec((BR, D), lambda i: (i, 0)),
            pl.BlockSpec((NC, BR, 1), lambda i: (0, i, 0)),
        ],
        out_specs=[
            pl.BlockSpec((BR, D), lambda i: (i, 0)),
            pl.BlockSpec((BR, 1), lambda i: (i, 0)),
        ],
        out_shape=[
            jax.ShapeDtypeStruct((NP, D), jnp.float32),
            jax.ShapeDtypeStruct((NP, 1), jnp.float32),
        ],
    )(h, degp_col)


def _fin_body(tp_ref, dinv_ref, b_ref, gamma_ref, beta_ref, o_ref):
    s = dinv_ref[...] * (tp_ref[0] + tp_ref[1]) + b_ref[...]
    mu = jnp.mean(s, axis=-1, keepdims=True)
    var = jnp.mean((s - mu) * (s - mu), axis=-1, keepdims=True)
    y = (s - mu) * lax.rsqrt(var + 1e-5) * gamma_ref[...] + beta_ref[...]
    o_ref[...] = jnp.maximum(y, 0.0)


def _tc_finish(tp, dinv_col, b, gamma, beta):
    return pl.pallas_call(
        _fin_body,
        grid=(GRID,),
        in_specs=[
            pl.BlockSpec((NC, BR, D), lambda i: (0, i, 0)),
            pl.BlockSpec((BR, 1), lambda i: (i, 0)),
            pl.BlockSpec((D,), lambda i: (0,)),
            pl.BlockSpec((D,), lambda i: (0,)),
            pl.BlockSpec((D,), lambda i: (0,)),
        ],
        out_specs=pl.BlockSpec((BR, D), lambda i: (i, 0)),
        out_shape=jax.ShapeDtypeStruct((N, D), jnp.float32),
    )(tp, dinv_col, b, gamma, beta)


def kernel(x, edge_index, W, b, gamma, beta):
    src = edge_index[0]
    dst = edge_index[1]
    zvec = jnp.zeros((NP,), jnp.float32)
    zrow = jnp.zeros((RPT, D), jnp.float32)

    degp = _deg_call(dst, zvec)
    h = _tc_matmul(x, W)
    degp_col = degp.reshape(NC, NP, 1)
    g, dinv_col = _tc_scale(h, degp_col)
    tp = _agg_call(g, src, dst, zrow)
    return _tc_finish(tp, dinv_col, b, gamma, beta)


# speedup vs baseline: 1.1032x; 1.1032x over previous
import functools

import jax
import jax.numpy as jnp
from jax import lax
from jax.experimental import pallas as pl
from jax.experimental.pallas import tpu as pltpu
from jax.experimental.pallas import tpu_sc as plsc

N = 10000
E = 320000
D = 128
NP = 10240
NC = 2
NS = 16
NW = NC * NS
EPW = E // NW
CH = 80
NCHUNK = EPW // CH
CHB = 80
NCHB = EPW // CHB
TAILB = EPW - NCHB * CHB
RPT = NP // NS
NB = 4
NBB = 4


def _sc_mesh():
    return plsc.VectorSubcoreMesh(core_axis_name="c", subcore_axis_name="s")


def _deg_body(dst_hbm, zeros_hbm, degp_hbm,
              di0, di1, di2, di3, ones_v, acc_sh,
              is0, is1, is2, is3, ss0, ss1, ss2, ss3):
    cid = lax.axis_index("c")
    sid = lax.axis_index("s")
    wid = cid * NS + sid
    r0 = sid * RPT
    dib = (di0, di1, di2, di3)
    isem = (is0, is1, is2, is3)
    ssem = (ss0, ss1, ss2, ss3)
    for k in range(CH // 16):
        ones_v[pl.ds(k * 16, 16)] = jnp.ones((16,), jnp.float32)
    pltpu.sync_copy(zeros_hbm.at[pl.ds(r0, RPT)], acc_sh.at[pl.ds(r0, RPT)])
    plsc.subcore_barrier()

    base = wid * EPW

    def group(k, carry):
        c0 = k * NB
        idescs = [
            pltpu.async_copy(
                dst_hbm.at[pl.ds(pl.multiple_of(base + (c0 + b) * CH, 8), CH)],
                dib[b], isem[b])
            for b in range(NB)
        ]
        sdescs = []
        for b in range(NB):
            idescs[b].wait()
            sdescs.append(
                pltpu.async_copy(ones_v, acc_sh.at[dib[b]], ssem[b], add=True))
        for d in sdescs:
            d.wait()
        return carry

    lax.fori_loop(0, NCHUNK // NB, group, 0)
    for c in range(NCHUNK - NCHUNK % NB, NCHUNK):
        off = pl.multiple_of(base + c * CH, 8)
        pltpu.async_copy(dst_hbm.at[pl.ds(off, CH)], di0, is0).wait()
        pltpu.async_copy(ones_v, acc_sh.at[di0], ss0, add=True).wait()
    plsc.subcore_barrier()
    pltpu.sync_copy(acc_sh.at[pl.ds(r0, RPT)], degp_hbm.at[cid, pl.ds(r0, RPT)])


_deg_call = functools.partial(
    pl.kernel,
    out_type=jax.ShapeDtypeStruct((NC, NP), jnp.float32),
    mesh=_sc_mesh(),
    scratch_types=(
        [pltpu.VMEM((CH,), jnp.int32) for _ in range(4)]
        + [pltpu.VMEM((CH,), jnp.float32)]
        + [pltpu.VMEM_SHARED((NP,), jnp.float32)]
        + [pltpu.SemaphoreType.DMA for _ in range(8)]
    ),
)(_deg_body)


def _agg_body(g_hbm, src_hbm, dst_hbm, zrow_hbm, tp_hbm, *rest):
    sib = rest[0:NBB]
    dib = rest[NBB:2 * NBB]
    sit, dit = rest[2 * NBB], rest[2 * NBB + 1]
    rows = rest[2 * NBB + 2:3 * NBB + 2]
    acc_sh = rest[3 * NBB + 2]
    isem = rest[3 * NBB + 3:4 * NBB + 3]
    gsem = rest[4 * NBB + 3:5 * NBB + 3]
    ssem = rest[5 * NBB + 3:6 * NBB + 3]
    cid = lax.axis_index("c")
    sid = lax.axis_index("s")
    wid = cid * NS + sid
    r0 = sid * RPT

    @pl.when(jnp.logical_and(cid == 0, sid < NS - 1))
    def _():
        pltpu.sync_copy(g_hbm.at[pl.ds(r0, RPT), :], acc_sh.at[pl.ds(r0, RPT), :])

    @pl.when(jnp.logical_and(cid == 0, sid == NS - 1))
    def _():
        pltpu.sync_copy(g_hbm.at[pl.ds(r0, N - r0), :],
                        acc_sh.at[pl.ds(r0, N - r0), :])
        pltpu.sync_copy(zrow_hbm.at[pl.ds(0, NP - N), :],
                        acc_sh.at[pl.ds(N, NP - N), :])

    @pl.when(cid != 0)
    def _():
        pltpu.sync_copy(zrow_hbm, acc_sh.at[pl.ds(r0, RPT), :])

    plsc.subcore_barrier()

    base = wid * EPW

    def drain_scatter(b):
        pltpu.make_async_copy(rows[b], acc_sh.at[dib[b]], ssem[b]).wait()

    def group(k, carry):
        c0 = k * NBB
        idescs = []
        for b in range(NBB):
            @pl.when(k > 0)
            def _():
                drain_scatter(b)
            off = pl.multiple_of(base + (c0 + b) * CHB, 8)
            idescs.append(
                (pltpu.async_copy(src_hbm.at[pl.ds(off, CHB)], sib[b], isem[b]),
                 pltpu.async_copy(dst_hbm.at[pl.ds(off, CHB)], dib[b], isem[b])))
        gdescs = []
        for b in range(NBB):
            idescs[b][0].wait()
            idescs[b][1].wait()
            gdescs.append(
                pltpu.async_copy(g_hbm.at[sib[b]], rows[b], gsem[b]))
        for b in range(NBB):
            gdescs[b].wait()
            pltpu.async_copy(rows[b], acc_sh.at[dib[b]], ssem[b], add=True)
        return carry

    lax.fori_loop(0, NCHB // NBB, group, 0)
    for b in range(NBB):
        drain_scatter(b)
    for c in range((NCHB // NBB) * NBB, NCHB):
        off = pl.multiple_of(base + c * CHB, 8)
        pltpu.async_copy(src_hbm.at[pl.ds(off, CHB)], sib[0], isem[0]).wait()
        pltpu.async_copy(dst_hbm.at[pl.ds(off, CHB)], dib[0], isem[0]).wait()
        pltpu.async_copy(g_hbm.at[sib[0]], rows[0], gsem[0]).wait()
        pltpu.sync_copy(rows[0], acc_sh.at[dib[0]], add=True)
    if TAILB:
        toff = pl.multiple_of(base + NCHB * CHB, 8)
        pltpu.async_copy(src_hbm.at[pl.ds(toff, TAILB)], sit, isem[0]).wait()
        pltpu.async_copy(dst_hbm.at[pl.ds(toff, TAILB)], dit, isem[0]).wait()
        pltpu.async_copy(g_hbm.at[sit], rows[0].at[pl.ds(0, TAILB), :],
                         gsem[0]).wait()
        pltpu.sync_copy(rows[0].at[pl.ds(0, TAILB), :], acc_sh.at[dit],
                        add=True)

    plsc.subcore_barrier()
    pltpu.sync_copy(acc_sh.at[pl.ds(r0, RPT), :],
                    tp_hbm.at[cid, pl.ds(r0, RPT), :])


_agg_call = functools.partial(
    pl.kernel,
    out_type=jax.ShapeDtypeStruct((NC, NP, D), jnp.float32),
    mesh=_sc_mesh(),
    scratch_types=(
        [pltpu.VMEM((CHB,), jnp.int32) for _ in range(2 * NBB)]
        + [pltpu.VMEM((max(TAILB, 8),), jnp.int32) for _ in range(2)]
        + [pltpu.VMEM((CHB, D), jnp.float32) for _ in range(NBB)]
        + [pltpu.VMEM_SHARED((NP, D), jnp.float32)]
        + [pltpu.SemaphoreType.DMA for _ in range(3 * NBB)]
    ),
)(_agg_body)


BR = 2000
GRID = N // BR


def _mm_body(x_ref, w_ref, h_ref):
    h_ref[...] = jnp.dot(x_ref[...], w_ref[...],
                         preferred_element_type=jnp.float32)


def _tc_matmul(x, W):
    return pl.pallas_call(
        _mm_body,
        grid=(GRID,),
        in_specs=[
            pl.BlockSpec((BR, D), lambda i: (i, 0)),
            pl.BlockSpec((D, D), lambda i: (0, 0)),
        ],
        out_specs=pl.BlockSpec((BR, D), lambda i: (i, 0)),
        out_shape=jax.ShapeDtypeStruct((NP, D), jnp.float32),
    )(x, W)


def _lin_body(h_ref, degp_ref, g_ref, dinv_ref):
    deg = jnp.sum(degp_ref[...], axis=0) + 1.0
    dinv = lax.rsqrt(deg)
    g_ref[...] = h_ref[...] * dinv
    dinv_ref[...] = dinv


def _tc_scale(h, degp_col):
    return pl.pallas_call(
        _lin_body,
        grid=(GRID,),
        in_specs=[
            pl.BlockSpec((BR, D), lambda i: (i, 0)),
            pl.BlockSpec((NC, BR, 1), lambda i: (0, i, 0)),
        ],
        out_specs=[
            pl.BlockSpec((BR, D), lambda i: (i, 0)),
            pl.BlockSpec((BR, 1), lambda i: (i, 0)),
        ],
        out_shape=[
            jax.ShapeDtypeStruct((NP, D), jnp.float32),
            jax.ShapeDtypeStruct((NP, 1), jnp.float32),
        ],
    )(h, degp_col)


def _fin_body(tp_ref, dinv_ref, b_ref, gamma_ref, beta_ref, o_ref):
    s = dinv_ref[...] * (tp_ref[0] + tp_ref[1]) + b_ref[...]
    mu = jnp.mean(s, axis=-1, keepdims=True)
    var = jnp.mean((s - mu) * (s - mu), axis=-1, keepdims=True)
    y = (s - mu) * lax.rsqrt(var + 1e-5) * gamma_ref[...] + beta_ref[...]
    o_ref[...] = jnp.maximum(y, 0.0)


def _tc_finish(tp, dinv_col, b, gamma, beta):
    return pl.pallas_call(
        _fin_body,
        grid=(GRID,),
        in_specs=[
            pl.BlockSpec((NC, BR, D), lambda i: (0, i, 0)),
            pl.BlockSpec((BR, 1), lambda i: (i, 0)),
            pl.BlockSpec((D,), lambda i: (0,)),
            pl.BlockSpec((D,), lambda i: (0,)),
            pl.BlockSpec((D,), lambda i: (0,)),
        ],
        out_specs=pl.BlockSpec((BR, D), lambda i: (i, 0)),
        out_shape=jax.ShapeDtypeStruct((N, D), jnp.float32),
    )(tp, dinv_col, b, gamma, beta)


def kernel(x, edge_index, W, b, gamma, beta):
    src = edge_index[0]
    dst = edge_index[1]
    zvec = jnp.zeros((NP,), jnp.float32)
    zrow = jnp.zeros((RPT, D), jnp.float32)

    degp = _deg_call(dst, zvec)
    h = _tc_matmul(x, W)
    degp_col = degp.reshape(NC, NP, 1)
    g, dinv_col = _tc_scale(h, degp_col)
    tp = _agg_call(g, src, dst, zrow)
    return _tc_finish(tp, dinv_col, b, gamma, beta)


# speedup vs baseline: 1.1059x; 1.0024x over previous
import functools

import jax
import jax.numpy as jnp
from jax import lax
from jax.experimental import pallas as pl
from jax.experimental.pallas import tpu as pltpu
from jax.experimental.pallas import tpu_sc as plsc

N = 10000
E = 320000
D = 128
NP = 10240
NC = 2
NS = 16
NW = NC * NS
EPW = E // NW
CH = 80
NCHUNK = EPW // CH
CHB = 80
NCHB = EPW // CHB
TAILB = EPW - NCHB * CHB
RPT = NP // NS
NB = 4
NBB = 4


def _sc_mesh():
    return plsc.VectorSubcoreMesh(core_axis_name="c", subcore_axis_name="s")


def _deg_body(dst_hbm, zeros_hbm, degp_hbm,
              di0, di1, di2, di3, ones_v, acc_sh,
              is0, is1, is2, is3, ss0, ss1, ss2, ss3):
    cid = lax.axis_index("c")
    sid = lax.axis_index("s")
    wid = cid * NS + sid
    r0 = sid * RPT
    dib = (di0, di1, di2, di3)
    isem = (is0, is1, is2, is3)
    ssem = (ss0, ss1, ss2, ss3)
    for k in range(CH // 16):
        ones_v[pl.ds(k * 16, 16)] = jnp.ones((16,), jnp.float32)
    pltpu.sync_copy(zeros_hbm.at[pl.ds(r0, RPT)], acc_sh.at[pl.ds(r0, RPT)])
    plsc.subcore_barrier()

    base = wid * EPW

    def drain(b):
        pltpu.make_async_copy(ones_v, acc_sh.at[dib[b]], ssem[b]).wait()

    def group(k, carry):
        c0 = k * NB
        idescs = []
        for b in range(NB):
            @pl.when(k > 0)
            def _():
                drain(b)
            off = pl.multiple_of(base + (c0 + b) * CH, 8)
            idescs.append(
                pltpu.async_copy(dst_hbm.at[pl.ds(off, CH)], dib[b], isem[b]))
        for b in range(NB):
            idescs[b].wait()
            pltpu.async_copy(ones_v, acc_sh.at[dib[b]], ssem[b], add=True)
        return carry

    lax.fori_loop(0, NCHUNK // NB, group, 0)
    for b in range(NB):
        drain(b)
    for c in range(NCHUNK - NCHUNK % NB, NCHUNK):
        off = pl.multiple_of(base + c * CH, 8)
        pltpu.async_copy(dst_hbm.at[pl.ds(off, CH)], di0, is0).wait()
        pltpu.async_copy(ones_v, acc_sh.at[di0], ss0, add=True).wait()
    plsc.subcore_barrier()
    pltpu.sync_copy(acc_sh.at[pl.ds(r0, RPT)], degp_hbm.at[cid, pl.ds(r0, RPT)])


_deg_call = functools.partial(
    pl.kernel,
    out_type=jax.ShapeDtypeStruct((NC, NP), jnp.float32),
    mesh=_sc_mesh(),
    scratch_types=(
        [pltpu.VMEM((CH,), jnp.int32) for _ in range(4)]
        + [pltpu.VMEM((CH,), jnp.float32)]
        + [pltpu.VMEM_SHARED((NP,), jnp.float32)]
        + [pltpu.SemaphoreType.DMA for _ in range(8)]
    ),
)(_deg_body)


def _agg_body(g_hbm, src_hbm, dst_hbm, zrow_hbm, tp_hbm, *rest):
    sib = rest[0:NBB]
    dib = rest[NBB:2 * NBB]
    sit, dit = rest[2 * NBB], rest[2 * NBB + 1]
    rows = rest[2 * NBB + 2:3 * NBB + 2]
    acc_sh = rest[3 * NBB + 2]
    isem = rest[3 * NBB + 3:4 * NBB + 3]
    gsem = rest[4 * NBB + 3:5 * NBB + 3]
    ssem = rest[5 * NBB + 3:6 * NBB + 3]
    cid = lax.axis_index("c")
    sid = lax.axis_index("s")
    wid = cid * NS + sid
    r0 = sid * RPT

    pltpu.sync_copy(zrow_hbm, acc_sh.at[pl.ds(r0, RPT), :])

    plsc.subcore_barrier()

    base = wid * EPW

    def drain_scatter(b):
        pltpu.make_async_copy(rows[b], acc_sh.at[dib[b]], ssem[b]).wait()

    def group(k, carry):
        c0 = k * NBB
        idescs = []
        for b in range(NBB):
            @pl.when(k > 0)
            def _():
                drain_scatter(b)
            off = pl.multiple_of(base + (c0 + b) * CHB, 8)
            idescs.append(
                (pltpu.async_copy(src_hbm.at[pl.ds(off, CHB)], sib[b], isem[b]),
                 pltpu.async_copy(dst_hbm.at[pl.ds(off, CHB)], dib[b], isem[b])))
        gdescs = []
        for b in range(NBB):
            idescs[b][0].wait()
            idescs[b][1].wait()
            gdescs.append(
                pltpu.async_copy(g_hbm.at[sib[b]], rows[b], gsem[b]))
        for b in range(NBB):
            gdescs[b].wait()
            pltpu.async_copy(rows[b], acc_sh.at[dib[b]], ssem[b], add=True)
        return carry

    lax.fori_loop(0, NCHB // NBB, group, 0)
    for b in range(NBB):
        drain_scatter(b)
    for c in range((NCHB // NBB) * NBB, NCHB):
        off = pl.multiple_of(base + c * CHB, 8)
        pltpu.async_copy(src_hbm.at[pl.ds(off, CHB)], sib[0], isem[0]).wait()
        pltpu.async_copy(dst_hbm.at[pl.ds(off, CHB)], dib[0], isem[0]).wait()
        pltpu.async_copy(g_hbm.at[sib[0]], rows[0], gsem[0]).wait()
        pltpu.sync_copy(rows[0], acc_sh.at[dib[0]], add=True)
    if TAILB:
        toff = pl.multiple_of(base + NCHB * CHB, 8)
        pltpu.async_copy(src_hbm.at[pl.ds(toff, TAILB)], sit, isem[0]).wait()
        pltpu.async_copy(dst_hbm.at[pl.ds(toff, TAILB)], dit, isem[0]).wait()
        pltpu.async_copy(g_hbm.at[sit], rows[0].at[pl.ds(0, TAILB), :],
                         gsem[0]).wait()
        pltpu.sync_copy(rows[0].at[pl.ds(0, TAILB), :], acc_sh.at[dit],
                        add=True)

    plsc.subcore_barrier()
    pltpu.sync_copy(acc_sh.at[pl.ds(r0, RPT), :],
                    tp_hbm.at[cid, pl.ds(r0, RPT), :])


_agg_call = functools.partial(
    pl.kernel,
    out_type=jax.ShapeDtypeStruct((NC, NP, D), jnp.float32),
    mesh=_sc_mesh(),
    scratch_types=(
        [pltpu.VMEM((CHB,), jnp.int32) for _ in range(2 * NBB)]
        + [pltpu.VMEM((max(TAILB, 8),), jnp.int32) for _ in range(2)]
        + [pltpu.VMEM((CHB, D), jnp.float32) for _ in range(NBB)]
        + [pltpu.VMEM_SHARED((NP, D), jnp.float32)]
        + [pltpu.SemaphoreType.DMA for _ in range(3 * NBB)]
    ),
)(_agg_body)


BR = 2000
GRID = N // BR


def _mm_body(x_ref, w_ref, h_ref):
    h_ref[...] = jnp.dot(x_ref[...], w_ref[...],
                         preferred_element_type=jnp.float32)


def _tc_matmul(x, W):
    return pl.pallas_call(
        _mm_body,
        grid=(GRID,),
        in_specs=[
            pl.BlockSpec((BR, D), lambda i: (i, 0)),
            pl.BlockSpec((D, D), lambda i: (0, 0)),
        ],
        out_specs=pl.BlockSpec((BR, D), lambda i: (i, 0)),
        out_shape=jax.ShapeDtypeStruct((NP, D), jnp.float32),
    )(x, W)


def _lin_body(h_ref, degp_ref, g_ref, dinv_ref):
    deg = jnp.sum(degp_ref[...], axis=0) + 1.0
    dinv = lax.rsqrt(deg)
    g_ref[...] = h_ref[...] * dinv
    dinv_ref[...] = dinv


def _tc_scale(h, degp_col):
    return pl.pallas_call(
        _lin_body,
        grid=(GRID,),
        in_specs=[
            pl.BlockSpec((BR, D), lambda i: (i, 0)),
            pl.BlockSpec((NC, BR, 1), lambda i: (0, i, 0)),
        ],
        out_specs=[
            pl.BlockSpec((BR, D), lambda i: (i, 0)),
            pl.BlockSpec((BR, 1), lambda i: (i, 0)),
        ],
        out_shape=[
            jax.ShapeDtypeStruct((NP, D), jnp.float32),
            jax.ShapeDtypeStruct((NP, 1), jnp.float32),
        ],
    )(h, degp_col)


def _fin_body(tp_ref, g_ref, dinv_ref, b_ref, gamma_ref, beta_ref, o_ref):
    s = dinv_ref[...] * (tp_ref[0] + tp_ref[1] + g_ref[...]) + b_ref[...]
    mu = jnp.mean(s, axis=-1, keepdims=True)
    var = jnp.mean((s - mu) * (s - mu), axis=-1, keepdims=True)
    y = (s - mu) * lax.rsqrt(var + 1e-5) * gamma_ref[...] + beta_ref[...]
    o_ref[...] = jnp.maximum(y, 0.0)


def _tc_finish(tp, g, dinv_col, b, gamma, beta):
    return pl.pallas_call(
        _fin_body,
        grid=(GRID,),
        in_specs=[
            pl.BlockSpec((NC, BR, D), lambda i: (0, i, 0)),
            pl.BlockSpec((BR, D), lambda i: (i, 0)),
            pl.BlockSpec((BR, 1), lambda i: (i, 0)),
            pl.BlockSpec((D,), lambda i: (0,)),
            pl.BlockSpec((D,), lambda i: (0,)),
            pl.BlockSpec((D,), lambda i: (0,)),
        ],
        out_specs=pl.BlockSpec((BR, D), lambda i: (i, 0)),
        out_shape=jax.ShapeDtypeStruct((N, D), jnp.float32),
    )(tp, g, dinv_col, b, gamma, beta)


def kernel(x, edge_index, W, b, gamma, beta):
    src = edge_index[0]
    dst = edge_index[1]
    zvec = jnp.zeros((NP,), jnp.float32)
    zrow = jnp.zeros((RPT, D), jnp.float32)

    degp = _deg_call(dst, zvec)
    h = _tc_matmul(x, W)
    degp_col = degp.reshape(NC, NP, 1)
    g, dinv_col = _tc_scale(h, degp_col)
    tp = _agg_call(g, src, dst, zrow)
    return _tc_finish(tp, g, dinv_col, b, gamma, beta)


# speedup vs baseline: 1.1774x; 1.0647x over previous
import functools

import jax
import jax.numpy as jnp
from jax import lax
from jax.experimental import pallas as pl
from jax.experimental.pallas import tpu as pltpu
from jax.experimental.pallas import tpu_sc as plsc

N = 10000
E = 320000
D = 128
NP = 10240
NC = 2
NS = 16
NW = NC * NS
EPW = E // NW
CH = 80
NCHUNK = EPW // CH
CHB = 80
NCHB = EPW // CHB
TAILB = EPW - NCHB * CHB
RPT = NP // NS
NB = 4
NBB = 4


def _sc_mesh():
    return plsc.VectorSubcoreMesh(core_axis_name="c", subcore_axis_name="s")


def _split_body(ei_ref, s_ref, d_ref):
    s_ref[...] = ei_ref[0, :]
    d_ref[...] = ei_ref[1, :]


def _tc_split(edge_index):
    return pl.pallas_call(
        _split_body,
        in_specs=[pl.BlockSpec((2, E), lambda: (0, 0))],
        out_specs=[
            pl.BlockSpec((E,), lambda: (0,)),
            pl.BlockSpec((E,), lambda: (0,)),
        ],
        out_shape=[
            jax.ShapeDtypeStruct((E,), jnp.int32),
            jax.ShapeDtypeStruct((E,), jnp.int32),
        ],
    )(edge_index)


def _deg_body(dst_hbm, zeros_hbm, degp_hbm,
              di0, di1, di2, di3, ones_v, acc_sh,
              is0, is1, is2, is3, ss0, ss1, ss2, ss3):
    cid = lax.axis_index("c")
    sid = lax.axis_index("s")
    wid = cid * NS + sid
    r0 = sid * RPT
    dib = (di0, di1, di2, di3)
    isem = (is0, is1, is2, is3)
    ssem = (ss0, ss1, ss2, ss3)
    for k in range(CH // 16):
        ones_v[pl.ds(k * 16, 16)] = jnp.ones((16,), jnp.float32)
    pltpu.sync_copy(zeros_hbm.at[pl.ds(r0, RPT)], acc_sh.at[pl.ds(r0, RPT)])
    plsc.subcore_barrier()

    base = wid * EPW

    def drain(b):
        pltpu.make_async_copy(ones_v, acc_sh.at[dib[b]], ssem[b]).wait()

    def group(k, carry):
        c0 = k * NB
        idescs = []
        for b in range(NB):
            @pl.when(k > 0)
            def _():
                drain(b)
            off = pl.multiple_of(base + (c0 + b) * CH, 8)
            idescs.append(
                pltpu.async_copy(dst_hbm.at[pl.ds(off, CH)], dib[b], isem[b]))
        for b in range(NB):
            idescs[b].wait()
            pltpu.async_copy(ones_v, acc_sh.at[dib[b]], ssem[b], add=True)
        return carry

    lax.fori_loop(0, NCHUNK // NB, group, 0)
    for b in range(NB):
        drain(b)
    for c in range(NCHUNK - NCHUNK % NB, NCHUNK):
        off = pl.multiple_of(base + c * CH, 8)
        pltpu.async_copy(dst_hbm.at[pl.ds(off, CH)], di0, is0).wait()
        pltpu.async_copy(ones_v, acc_sh.at[di0], ss0, add=True).wait()
    plsc.subcore_barrier()
    pltpu.sync_copy(acc_sh.at[pl.ds(r0, RPT)], degp_hbm.at[cid, pl.ds(r0, RPT)])


_deg_call = functools.partial(
    pl.kernel,
    out_type=jax.ShapeDtypeStruct((NC, NP), jnp.float32),
    mesh=_sc_mesh(),
    scratch_types=(
        [pltpu.VMEM((CH,), jnp.int32) for _ in range(4)]
        + [pltpu.VMEM((CH,), jnp.float32)]
        + [pltpu.VMEM_SHARED((NP,), jnp.float32)]
        + [pltpu.SemaphoreType.DMA for _ in range(8)]
    ),
)(_deg_body)


def _agg_body(g_hbm, src_hbm, dst_hbm, zrow_hbm, tp_hbm, *rest):
    sib = rest[0:NBB]
    dib = rest[NBB:2 * NBB]
    sit, dit = rest[2 * NBB], rest[2 * NBB + 1]
    rows = rest[2 * NBB + 2:3 * NBB + 2]
    acc_sh = rest[3 * NBB + 2]
    isem = rest[3 * NBB + 3:4 * NBB + 3]
    gsem = rest[4 * NBB + 3:5 * NBB + 3]
    ssem = rest[5 * NBB + 3:6 * NBB + 3]
    cid = lax.axis_index("c")
    sid = lax.axis_index("s")
    wid = cid * NS + sid
    r0 = sid * RPT

    pltpu.sync_copy(zrow_hbm, acc_sh.at[pl.ds(r0, RPT), :])

    plsc.subcore_barrier()

    base = wid * EPW

    def drain_scatter(b):
        pltpu.make_async_copy(rows[b], acc_sh.at[dib[b]], ssem[b]).wait()

    def group(k, carry):
        c0 = k * NBB
        idescs = []
        for b in range(NBB):
            @pl.when(k > 0)
            def _():
                drain_scatter(b)
            off = pl.multiple_of(base + (c0 + b) * CHB, 8)
            idescs.append(
                (pltpu.async_copy(src_hbm.at[pl.ds(off, CHB)], sib[b], isem[b]),
                 pltpu.async_copy(dst_hbm.at[pl.ds(off, CHB)], dib[b], isem[b])))
        gdescs = []
        for b in range(NBB):
            idescs[b][0].wait()
            idescs[b][1].wait()
            gdescs.append(
                pltpu.async_copy(g_hbm.at[sib[b]], rows[b], gsem[b]))
        for b in range(NBB):
            gdescs[b].wait()
            pltpu.async_copy(rows[b], acc_sh.at[dib[b]], ssem[b], add=True)
        return carry

    lax.fori_loop(0, NCHB // NBB, group, 0)
    for b in range(NBB):
        drain_scatter(b)
    for c in range((NCHB // NBB) * NBB, NCHB):
        off = pl.multiple_of(base + c * CHB, 8)
        pltpu.async_copy(src_hbm.at[pl.ds(off, CHB)], sib[0], isem[0]).wait()
        pltpu.async_copy(dst_hbm.at[pl.ds(off, CHB)], dib[0], isem[0]).wait()
        pltpu.async_copy(g_hbm.at[sib[0]], rows[0], gsem[0]).wait()
        pltpu.sync_copy(rows[0], acc_sh.at[dib[0]], add=True)
    if TAILB:
        toff = pl.multiple_of(base + NCHB * CHB, 8)
        pltpu.async_copy(src_hbm.at[pl.ds(toff, TAILB)], sit, isem[0]).wait()
        pltpu.async_copy(dst_hbm.at[pl.ds(toff, TAILB)], dit, isem[0]).wait()
        pltpu.async_copy(g_hbm.at[sit], rows[0].at[pl.ds(0, TAILB), :],
                         gsem[0]).wait()
        pltpu.sync_copy(rows[0].at[pl.ds(0, TAILB), :], acc_sh.at[dit],
                        add=True)

    plsc.subcore_barrier()
    pltpu.sync_copy(acc_sh.at[pl.ds(r0, RPT), :],
                    tp_hbm.at[cid, pl.ds(r0, RPT), :])


_agg_call = functools.partial(
    pl.kernel,
    out_type=jax.ShapeDtypeStruct((NC, NP, D), jnp.float32),
    mesh=_sc_mesh(),
    scratch_types=(
        [pltpu.VMEM((CHB,), jnp.int32) for _ in range(2 * NBB)]
        + [pltpu.VMEM((max(TAILB, 8),), jnp.int32) for _ in range(2)]
        + [pltpu.VMEM((CHB, D), jnp.float32) for _ in range(NBB)]
        + [pltpu.VMEM_SHARED((NP, D), jnp.float32)]
        + [pltpu.SemaphoreType.DMA for _ in range(3 * NBB)]
    ),
)(_agg_body)


BR = 2000
GRID = N // BR


def _mm_body(x_ref, w_ref, h_ref):
    h_ref[...] = jnp.dot(x_ref[...], w_ref[...],
                         preferred_element_type=jnp.float32)


def _tc_matmul(x, W):
    return pl.pallas_call(
        _mm_body,
        grid=(GRID,),
        in_specs=[
            pl.BlockSpec((BR, D), lambda i: (i, 0)),
            pl.BlockSpec((D, D), lambda i: (0, 0)),
        ],
        out_specs=pl.BlockSpec((BR, D), lambda i: (i, 0)),
        out_shape=jax.ShapeDtypeStruct((NP, D), jnp.float32),
    )(x, W)


def _lin_body(h_ref, degp_ref, g_ref, dinv_ref):
    deg = jnp.sum(degp_ref[...], axis=0) + 1.0
    dinv = lax.rsqrt(deg)
    g_ref[...] = h_ref[...] * dinv
    dinv_ref[...] = dinv


def _tc_scale(h, degp_col):
    return pl.pallas_call(
        _lin_body,
        grid=(GRID,),
        in_specs=[
            pl.BlockSpec((BR, D), lambda i: (i, 0)),
            pl.BlockSpec((NC, BR, 1), lambda i: (0, i, 0)),
        ],
        out_specs=[
            pl.BlockSpec((BR, D), lambda i: (i, 0)),
            pl.BlockSpec((BR, 1), lambda i: (i, 0)),
        ],
        out_shape=[
            jax.ShapeDtypeStruct((NP, D), jnp.float32),
            jax.ShapeDtypeStruct((NP, 1), jnp.float32),
        ],
    )(h, degp_col)


def _fin_body(tp_ref, g_ref, dinv_ref, b_ref, gamma_ref, beta_ref, o_ref):
    s = dinv_ref[...] * (tp_ref[0] + tp_ref[1] + g_ref[...]) + b_ref[...]
    mu = jnp.mean(s, axis=-1, keepdims=True)
    var = jnp.mean((s - mu) * (s - mu), axis=-1, keepdims=True)
    y = (s - mu) * lax.rsqrt(var + 1e-5) * gamma_ref[...] + beta_ref[...]
    o_ref[...] = jnp.maximum(y, 0.0)


def _tc_finish(tp, g, dinv_col, b, gamma, beta):
    return pl.pallas_call(
        _fin_body,
        grid=(GRID,),
        in_specs=[
            pl.BlockSpec((NC, BR, D), lambda i: (0, i, 0)),
            pl.BlockSpec((BR, D), lambda i: (i, 0)),
            pl.BlockSpec((BR, 1), lambda i: (i, 0)),
            pl.BlockSpec((D,), lambda i: (0,)),
            pl.BlockSpec((D,), lambda i: (0,)),
            pl.BlockSpec((D,), lambda i: (0,)),
        ],
        out_specs=pl.BlockSpec((BR, D), lambda i: (i, 0)),
        out_shape=jax.ShapeDtypeStruct((N, D), jnp.float32),
    )(tp, g, dinv_col, b, gamma, beta)


def kernel(x, edge_index, W, b, gamma, beta):
    zvec = jnp.zeros((NP,), jnp.float32)
    zrow = jnp.zeros((RPT, D), jnp.float32)

    src, dst = _tc_split(edge_index)
    degp = _deg_call(dst, zvec)
    h = _tc_matmul(x, W)
    degp_col = degp.reshape(NC, NP, 1)
    g, dinv_col = _tc_scale(h, degp_col)
    tp = _agg_call(g, src, dst, zrow)
    return _tc_finish(tp, g, dinv_col, b, gamma, beta)
